# Initial kernel scaffold; baseline (speedup 1.0000x reference)
import jax
import jax.numpy as jnp
from jax.experimental import pallas as pl


def kernel(node_feat, edge_index, edge_feat, W_fc, W_edge, attn_l, attn_r):
    raise NotImplementedError("write your pallas kernel here")


# speedup vs baseline: 4.3535x; 4.3535x over previous
import functools

import jax
import jax.numpy as jnp
from jax import lax
from jax.experimental import pallas as pl
from jax.experimental.pallas import tpu as pltpu
from jax.experimental.pallas import tpu_sc as plsc

N = 10000
E = 320000
D_IN = 128
D_OUT = 128
C = 16

NC = 2
NS = 16
NW = NC * NS
CHUNK = 128
NCHUNK = E // CHUNK
ROWS_MAIN = 624
TAIL_BASE = NS * ROWS_MAIN
TAIL_ROWS = N - TAIL_BASE
SLOPE = 0.2

_EB = 20000


def _tc_prep_body(nf, w3, a_l, a_r, ef_in, we, el, er, ef_out):
    i = pl.program_id(0)

    @pl.when(i == 0)
    def _():
        al_mat = jnp.sum(w3[...] * a_l[...][None], axis=-1)
        ar_mat = jnp.sum(w3[...] * a_r[...][None], axis=-1)
        nfv = nf[...]
        el[...] = jnp.dot(nfv, al_mat, preferred_element_type=jnp.float32)
        er[...] = jnp.dot(nfv, ar_mat, preferred_element_type=jnp.float32)

    ef_out[...] = jnp.dot(ef_in[...], we[...], preferred_element_type=jnp.float32)


def _tc_prep(node_feat, W3, al, ar, edge_feat, W_edge):
    return pl.pallas_call(
        _tc_prep_body,
        grid=(E // _EB,),
        in_specs=[
            pl.BlockSpec((N, D_IN), lambda i: (0, 0)),
            pl.BlockSpec((D_IN, C, D_OUT), lambda i: (0, 0, 0)),
            pl.BlockSpec((C, D_OUT), lambda i: (0, 0)),
            pl.BlockSpec((C, D_OUT), lambda i: (0, 0)),
            pl.BlockSpec((_EB, C), lambda i: (i, 0)),
            pl.BlockSpec((C, C), lambda i: (0, 0)),
        ],
        out_specs=[
            pl.BlockSpec((N, C), lambda i: (0, 0)),
            pl.BlockSpec((N, C), lambda i: (0, 0)),
            pl.BlockSpec((_EB, C), lambda i: (i, 0)),
        ],
        out_shape=[
            jax.ShapeDtypeStruct((N, C), jnp.float32),
            jax.ShapeDtypeStruct((N, C), jnp.float32),
            jax.ShapeDtypeStruct((E, C), jnp.float32),
        ],
    )(node_feat, W3, al, ar, edge_feat, W_edge)


_mesh = plsc.VectorSubcoreMesh(core_axis_name="c", subcore_axis_name="s")


@functools.partial(
    pl.kernel,
    out_type=(
        jax.ShapeDtypeStruct((E, C), jnp.float32),
        jax.ShapeDtypeStruct((NC, N, C), jnp.float32),
    ),
    mesh=_mesh,
    compiler_params=pltpu.CompilerParams(use_tc_tiling_on_sc=False),
    scratch_types=[
        pltpu.VMEM((CHUNK,), jnp.int32),
        pltpu.VMEM((CHUNK,), jnp.int32),
        pltpu.VMEM((CHUNK, C), jnp.float32),
        pltpu.VMEM((CHUNK, C), jnp.float32),
        pltpu.VMEM((CHUNK, C), jnp.float32),
        pltpu.VMEM((CHUNK, C), jnp.float32),
        pltpu.VMEM((ROWS_MAIN, C), jnp.float32),
        pltpu.VMEM_SHARED((N, C), jnp.float32),
        pltpu.SemaphoreType.DMA,
        pltpu.SemaphoreType.DMA,
    ],
)
def _sc_pass1(src_hbm, dst_hbm, el_hbm, er_hbm, ef_hbm,
              ex_hbm, spart_hbm,
              idx_s, idx_d, elb, erb, efb, exb, srow, s_sh, sem1, sem2):
    cid = lax.axis_index("c")
    sid = lax.axis_index("s")
    wid = sid * NC + cid

    def zero_body(j, _):
        srow[j] = jnp.zeros((C,), jnp.float32)
        return 0

    lax.fori_loop(0, ROWS_MAIN, zero_body, 0, unroll=8)
    pltpu.sync_copy(srow, s_sh.at[pl.ds(sid * ROWS_MAIN, ROWS_MAIN)])

    @pl.when(sid == 0)
    def _():
        pltpu.sync_copy(srow.at[pl.ds(0, TAIL_ROWS)],
                        s_sh.at[pl.ds(TAIL_BASE, TAIL_ROWS)])

    plsc.subcore_barrier()

    n_extra = NCHUNK % NW
    n_i = (NCHUNK // NW) + jnp.where(wid < n_extra, 1, 0)

    def chunk_body(i, _):
        base = (i * NW + wid) * CHUNK
        pltpu.sync_copy(src_hbm.at[pl.ds(base, CHUNK)], idx_s)
        pltpu.sync_copy(dst_hbm.at[pl.ds(base, CHUNK)], idx_d)
        cp1 = pltpu.async_copy(el_hbm.at[idx_s], elb, sem1)
        cp2 = pltpu.async_copy(er_hbm.at[idx_d], erb, sem2)
        pltpu.sync_copy(ef_hbm.at[pl.ds(base, CHUNK)], efb)
        cp1.wait()
        cp2.wait()

        def row_body(j, _):
            v = elb[j] + erb[j]
            v = jnp.where(v > 0, v, SLOPE * v)
            exb[j] = jnp.exp(v * efb[j])
            return 0

        lax.fori_loop(0, CHUNK, row_body, 0, unroll=8)
        pltpu.sync_copy(exb, ex_hbm.at[pl.ds(base, CHUNK)])
        pltpu.sync_copy(exb, s_sh.at[idx_d], add=True)
        return 0

    lax.fori_loop(0, n_i, chunk_body, 0)
    plsc.subcore_barrier()

    pltpu.sync_copy(s_sh.at[pl.ds(sid * ROWS_MAIN, ROWS_MAIN)], srow)
    pltpu.sync_copy(srow, spart_hbm.at[cid, pl.ds(sid * ROWS_MAIN, ROWS_MAIN)])

    @pl.when(sid == 0)
    def _():
        pltpu.sync_copy(s_sh.at[pl.ds(TAIL_BASE, TAIL_ROWS)],
                        srow.at[pl.ds(0, TAIL_ROWS)])
        pltpu.sync_copy(srow.at[pl.ds(0, TAIL_ROWS)],
                        spart_hbm.at[cid, pl.ds(TAIL_BASE, TAIL_ROWS)])


@functools.partial(
    pl.kernel,
    out_type=jax.ShapeDtypeStruct((E, C), jnp.float32),
    mesh=_mesh,
    compiler_params=pltpu.CompilerParams(use_tc_tiling_on_sc=False),
    scratch_types=[
        pltpu.VMEM((CHUNK,), jnp.int32),
        pltpu.VMEM((CHUNK, C), jnp.float32),
        pltpu.VMEM((CHUNK, C), jnp.float32),
        pltpu.VMEM((CHUNK, C), jnp.float32),
        pltpu.VMEM((CHUNK, C), jnp.float32),
        pltpu.SemaphoreType.DMA,
        pltpu.SemaphoreType.DMA,
    ],
)
def _sc_pass2(dst_hbm, ex_hbm, s0_hbm, s1_hbm, out_hbm,
              idx_d, exb, s0b, s1b, outb, sem1, sem2):
    cid = lax.axis_index("c")
    sid = lax.axis_index("s")
    wid = sid * NC + cid

    n_extra = NCHUNK % NW
    n_i = (NCHUNK // NW) + jnp.where(wid < n_extra, 1, 0)

    def chunk_body(i, _):
        base = (i * NW + wid) * CHUNK
        pltpu.sync_copy(dst_hbm.at[pl.ds(base, CHUNK)], idx_d)
        cp1 = pltpu.async_copy(s0_hbm.at[idx_d], s0b, sem1)
        cp2 = pltpu.async_copy(s1_hbm.at[idx_d], s1b, sem2)
        pltpu.sync_copy(ex_hbm.at[pl.ds(base, CHUNK)], exb)
        cp1.wait()
        cp2.wait()

        def row_body(j, _):
            outb[j] = exb[j] / (s0b[j] + s1b[j])
            return 0

        lax.fori_loop(0, CHUNK, row_body, 0, unroll=8)
        pltpu.sync_copy(outb, out_hbm.at[pl.ds(base, CHUNK)])
        return 0

    lax.fori_loop(0, n_i, chunk_body, 0)


def kernel(node_feat, edge_index, edge_feat, W_fc, W_edge, attn_l, attn_r):
    src = edge_index[0]
    dst = edge_index[1]
    W3 = W_fc.reshape(D_IN, C, D_OUT)
    al = attn_l.reshape(C, D_OUT)
    ar = attn_r.reshape(C, D_OUT)
    el, er, ef = _tc_prep(node_feat, W3, al, ar, edge_feat, W_edge)
    ex, spart = _sc_pass1(src, dst, el, er, ef)
    a = _sc_pass2(dst, ex, spart[0], spart[1])
    return a.reshape(E, C, 1)


# speedup vs baseline: 5.6114x; 1.2889x over previous
import functools

import jax
import jax.numpy as jnp
from jax import lax
from jax.experimental import pallas as pl
from jax.experimental.pallas import tpu as pltpu
from jax.experimental.pallas import tpu_sc as plsc

N = 10000
E = 320000
D_IN = 128
D_OUT = 128
C = 16

NC = 2
NS = 16
NW = NC * NS
CHUNK = 128
NCHUNK = E // CHUNK
CH_BASE = NCHUNK // NW
CH_EXTRA = NCHUNK % NW
PAIRS = (CH_BASE + CH_EXTRA + 1) // 2
ROWS_MAIN = 624
TAIL_BASE = NS * ROWS_MAIN
TAIL_ROWS = N - TAIL_BASE
SLOPE = 0.2

_EB = 20000


def _tc_prep_body(nf, w3, a_l, a_r, ef_in, we, el, er, ef_out):
    i = pl.program_id(0)

    @pl.when(i == 0)
    def _():
        al_mat = jnp.sum(w3[...] * a_l[...][None], axis=-1)
        ar_mat = jnp.sum(w3[...] * a_r[...][None], axis=-1)
        nfv = nf[...]
        el[...] = jnp.dot(nfv, al_mat, preferred_element_type=jnp.float32)
        er[...] = jnp.dot(nfv, ar_mat, preferred_element_type=jnp.float32)

    ef_out[...] = jnp.dot(ef_in[...], we[...], preferred_element_type=jnp.float32)


def _tc_prep(node_feat, W3, al, ar, edge_feat, W_edge):
    return pl.pallas_call(
        _tc_prep_body,
        grid=(E // _EB,),
        in_specs=[
            pl.BlockSpec((N, D_IN), lambda i: (0, 0)),
            pl.BlockSpec((D_IN, C, D_OUT), lambda i: (0, 0, 0)),
            pl.BlockSpec((C, D_OUT), lambda i: (0, 0)),
            pl.BlockSpec((C, D_OUT), lambda i: (0, 0)),
            pl.BlockSpec((_EB, C), lambda i: (i, 0)),
            pl.BlockSpec((C, C), lambda i: (0, 0)),
        ],
        out_specs=[
            pl.BlockSpec((N, C), lambda i: (0, 0)),
            pl.BlockSpec((N, C), lambda i: (0, 0)),
            pl.BlockSpec((_EB, C), lambda i: (i, 0)),
        ],
        out_shape=[
            jax.ShapeDtypeStruct((N, C), jnp.float32),
            jax.ShapeDtypeStruct((N, C), jnp.float32),
            jax.ShapeDtypeStruct((E, C), jnp.float32),
        ],
    )(node_feat, W3, al, ar, edge_feat, W_edge)


_mesh = plsc.VectorSubcoreMesh(core_axis_name="c", subcore_axis_name="s")


@functools.partial(
    pl.kernel,
    out_type=(
        jax.ShapeDtypeStruct((E, C), jnp.float32),
        jax.ShapeDtypeStruct((NC, N, C), jnp.float32),
    ),
    mesh=_mesh,
    compiler_params=pltpu.CompilerParams(use_tc_tiling_on_sc=False),
    scratch_types=[
        pltpu.VMEM((CH_BASE + 1, CHUNK), jnp.int32),
        pltpu.VMEM((CH_BASE + 1, CHUNK), jnp.int32),
        pltpu.VMEM((2, CHUNK, C), jnp.float32),
        pltpu.VMEM((2, CHUNK, C), jnp.float32),
        pltpu.VMEM((2, CHUNK, C), jnp.float32),
        pltpu.VMEM((2, CHUNK, C), jnp.float32),
        pltpu.VMEM((ROWS_MAIN, C), jnp.float32),
        pltpu.VMEM_SHARED((N, C), jnp.float32),
        pltpu.SemaphoreType.DMA,
        pltpu.SemaphoreType.DMA,
    ],
)
def _sc_pass1(src_hbm, dst_hbm, el_hbm, er_hbm, ef_hbm,
              ex_hbm, spart_hbm,
              idx_s, idx_d, elb, erb, efb, exb, srow, s_sh, sem0, sem1):
    cid = lax.axis_index("c")
    sid = lax.axis_index("s")
    wid = sid * NC + cid
    start = CH_BASE * wid + jnp.minimum(wid, CH_EXTRA)
    n_w = CH_BASE + jnp.where(wid < CH_EXTRA, 1, 0)

    def zero_body(j, _):
        srow[j] = jnp.zeros((C,), jnp.float32)
        return 0

    lax.fori_loop(0, ROWS_MAIN, zero_body, 0, unroll=8)
    pltpu.sync_copy(srow, s_sh.at[pl.ds(sid * ROWS_MAIN, ROWS_MAIN)])

    @pl.when(sid == 0)
    def _():
        pltpu.sync_copy(srow.at[pl.ds(0, TAIL_ROWS)],
                        s_sh.at[pl.ds(TAIL_BASE, TAIL_ROWS)])

    pltpu.sync_copy(src_hbm.at[pl.ds(start, CH_BASE)], idx_s.at[pl.ds(0, CH_BASE)])
    pltpu.sync_copy(dst_hbm.at[pl.ds(start, CH_BASE)], idx_d.at[pl.ds(0, CH_BASE)])

    @pl.when(wid < CH_EXTRA)
    def _():
        pltpu.sync_copy(src_hbm.at[pl.ds(start + CH_BASE, 1)],
                        idx_s.at[pl.ds(CH_BASE, 1)])
        pltpu.sync_copy(dst_hbm.at[pl.ds(start + CH_BASE, 1)],
                        idx_d.at[pl.ds(CH_BASE, 1)])

    plsc.subcore_barrier()

    def issue(j, slot, sem):
        base = (start + j) * CHUNK
        pltpu.async_copy(el_hbm.at[idx_s.at[j]], elb.at[slot], sem)
        pltpu.async_copy(er_hbm.at[idx_d.at[j]], erb.at[slot], sem)
        pltpu.async_copy(ef_hbm.at[pl.ds(base, CHUNK)], efb.at[slot], sem)

    def wait_in(slot, sem):
        pltpu.make_async_copy(el_hbm.at[idx_s.at[0]], elb.at[slot], sem).wait()
        pltpu.make_async_copy(er_hbm.at[idx_d.at[0]], erb.at[slot], sem).wait()
        pltpu.make_async_copy(ef_hbm.at[pl.ds(0, CHUNK)], efb.at[slot], sem).wait()

    def process(j, slot):
        def row_body(r, _):
            v = elb[slot, r] + erb[slot, r]
            v = jnp.where(v > 0, v, SLOPE * v)
            exb[slot, r] = jnp.exp(v * efb[slot, r])
            return 0

        lax.fori_loop(0, CHUNK, row_body, 0, unroll=8)
        pltpu.sync_copy(exb.at[slot], s_sh.at[idx_d.at[j]], add=True)
        pltpu.sync_copy(exb.at[slot], ex_hbm.at[pl.ds((start + j) * CHUNK, CHUNK)])

    issue(0, 0, sem0)

    def pair_body(p, _):
        i0 = 2 * p
        i1 = i0 + 1

        @pl.when(i0 < n_w)
        def _():
            @pl.when(i1 < n_w)
            def _():
                issue(i1, 1, sem1)

            wait_in(0, sem0)
            process(i0, 0)

        @pl.when(i1 < n_w)
        def _():
            @pl.when(i1 + 1 < n_w)
            def _():
                issue(i1 + 1, 0, sem0)

            wait_in(1, sem1)
            process(i1, 1)

        return 0

    lax.fori_loop(0, PAIRS, pair_body, 0)
    plsc.subcore_barrier()

    pltpu.sync_copy(s_sh.at[pl.ds(sid * ROWS_MAIN, ROWS_MAIN)], srow)
    pltpu.sync_copy(srow, spart_hbm.at[cid, pl.ds(sid * ROWS_MAIN, ROWS_MAIN)])

    @pl.when(sid == 0)
    def _():
        pltpu.sync_copy(s_sh.at[pl.ds(TAIL_BASE, TAIL_ROWS)],
                        srow.at[pl.ds(0, TAIL_ROWS)])
        pltpu.sync_copy(srow.at[pl.ds(0, TAIL_ROWS)],
                        spart_hbm.at[cid, pl.ds(TAIL_BASE, TAIL_ROWS)])


@functools.partial(
    pl.kernel,
    out_type=jax.ShapeDtypeStruct((E, C), jnp.float32),
    mesh=_mesh,
    compiler_params=pltpu.CompilerParams(use_tc_tiling_on_sc=False),
    scratch_types=[
        pltpu.VMEM((CH_BASE + 1, CHUNK), jnp.int32),
        pltpu.VMEM((2, CHUNK, C), jnp.float32),
        pltpu.VMEM((2, CHUNK, C), jnp.float32),
        pltpu.VMEM((2, CHUNK, C), jnp.float32),
        pltpu.VMEM((2, CHUNK, C), jnp.float32),
        pltpu.SemaphoreType.DMA,
        pltpu.SemaphoreType.DMA,
    ],
)
def _sc_pass2(dst_hbm, ex_hbm, s0_hbm, s1_hbm, out_hbm,
              idx_d, exb, s0b, s1b, outb, sem0, sem1):
    cid = lax.axis_index("c")
    sid = lax.axis_index("s")
    wid = sid * NC + cid
    start = CH_BASE * wid + jnp.minimum(wid, CH_EXTRA)
    n_w = CH_BASE + jnp.where(wid < CH_EXTRA, 1, 0)

    pltpu.sync_copy(dst_hbm.at[pl.ds(start, CH_BASE)], idx_d.at[pl.ds(0, CH_BASE)])

    @pl.when(wid < CH_EXTRA)
    def _():
        pltpu.sync_copy(dst_hbm.at[pl.ds(start + CH_BASE, 1)],
                        idx_d.at[pl.ds(CH_BASE, 1)])

    def issue(j, slot, sem):
        base = (start + j) * CHUNK
        pltpu.async_copy(s0_hbm.at[idx_d.at[j]], s0b.at[slot], sem)
        pltpu.async_copy(s1_hbm.at[idx_d.at[j]], s1b.at[slot], sem)
        pltpu.async_copy(ex_hbm.at[pl.ds(base, CHUNK)], exb.at[slot], sem)

    def wait_in(slot, sem):
        pltpu.make_async_copy(s0_hbm.at[idx_d.at[0]], s0b.at[slot], sem).wait()
        pltpu.make_async_copy(s1_hbm.at[idx_d.at[0]], s1b.at[slot], sem).wait()
        pltpu.make_async_copy(ex_hbm.at[pl.ds(0, CHUNK)], exb.at[slot], sem).wait()

    def process(j, slot):
        def row_body(r, _):
            outb[slot, r] = exb[slot, r] / (s0b[slot, r] + s1b[slot, r])
            return 0

        lax.fori_loop(0, CHUNK, row_body, 0, unroll=8)
        pltpu.sync_copy(outb.at[slot], out_hbm.at[pl.ds((start + j) * CHUNK, CHUNK)])

    issue(0, 0, sem0)

    def pair_body(p, _):
        i0 = 2 * p
        i1 = i0 + 1

        @pl.when(i0 < n_w)
        def _():
            @pl.when(i1 < n_w)
            def _():
                issue(i1, 1, sem1)

            wait_in(0, sem0)
            process(i0, 0)

        @pl.when(i1 < n_w)
        def _():
            @pl.when(i1 + 1 < n_w)
            def _():
                issue(i1 + 1, 0, sem0)

            wait_in(1, sem1)
            process(i1, 1)

        return 0

    lax.fori_loop(0, PAIRS, pair_body, 0)


def kernel(node_feat, edge_index, edge_feat, W_fc, W_edge, attn_l, attn_r):
    src2d = edge_index[0].reshape(NCHUNK, CHUNK)
    dst2d = edge_index[1].reshape(NCHUNK, CHUNK)
    W3 = W_fc.reshape(D_IN, C, D_OUT)
    al = attn_l.reshape(C, D_OUT)
    ar = attn_r.reshape(C, D_OUT)
    el, er, ef = _tc_prep(node_feat, W3, al, ar, edge_feat, W_edge)
    ex, spart = _sc_pass1(src2d, dst2d, el, er, ef)
    a = _sc_pass2(dst2d, ex, spart[0], spart[1])
    return a.reshape(E, C, 1)


# speedup vs baseline: 7.5666x; 1.3484x over previous
import functools

import jax
import jax.numpy as jnp
from jax import lax
from jax.experimental import pallas as pl
from jax.experimental.pallas import tpu as pltpu
from jax.experimental.pallas import tpu_sc as plsc

N = 10000
E = 320000
D_IN = 128
D_OUT = 128
C = 16

NC = 2
NS = 16
NW = NC * NS
CHUNK = 128
NCHUNK = E // CHUNK
CH_BASE = NCHUNK // NW
CH_EXTRA = NCHUNK % NW
PAIRS = (CH_BASE + CH_EXTRA + 1) // 2
ROWS_MAIN = 624
TAIL_BASE = NS * ROWS_MAIN
TAIL_ROWS = N - TAIL_BASE
SLOPE = 0.2

_EB = 32000


def _tc_prep_body(nf, w3, a_l, a_r, eft_in, we, el, er, eft_out):
    i = pl.program_id(0)

    @pl.when(i == 0)
    def _():
        al_mat = jnp.sum(w3[...] * a_l[...][None], axis=-1)
        ar_mat = jnp.sum(w3[...] * a_r[...][None], axis=-1)
        nfv = nf[...]
        el[...] = jnp.dot(nfv, al_mat, preferred_element_type=jnp.float32)
        er[...] = jnp.dot(nfv, ar_mat, preferred_element_type=jnp.float32)

    eft_out[...] = lax.dot_general(
        we[...], eft_in[...], (((0,), (0,)), ((), ())),
        preferred_element_type=jnp.float32)


def _tc_prep(node_feat, W3, al, ar, edge_feat_t, W_edge):
    return pl.pallas_call(
        _tc_prep_body,
        grid=(E // _EB,),
        in_specs=[
            pl.BlockSpec((N, D_IN), lambda i: (0, 0)),
            pl.BlockSpec((D_IN, C, D_OUT), lambda i: (0, 0, 0)),
            pl.BlockSpec((C, D_OUT), lambda i: (0, 0)),
            pl.BlockSpec((C, D_OUT), lambda i: (0, 0)),
            pl.BlockSpec((C, _EB), lambda i: (0, i)),
            pl.BlockSpec((C, C), lambda i: (0, 0)),
        ],
        out_specs=[
            pl.BlockSpec((N, C), lambda i: (0, 0)),
            pl.BlockSpec((N, C), lambda i: (0, 0)),
            pl.BlockSpec((C, _EB), lambda i: (0, i)),
        ],
        out_shape=[
            jax.ShapeDtypeStruct((N, C), jnp.float32),
            jax.ShapeDtypeStruct((N, C), jnp.float32),
            jax.ShapeDtypeStruct((C, E), jnp.float32),
        ],
    )(node_feat, W3, al, ar, edge_feat_t, W_edge)


_mesh = plsc.VectorSubcoreMesh(core_axis_name="c", subcore_axis_name="s")


@functools.partial(
    pl.kernel,
    out_type=(
        jax.ShapeDtypeStruct((E, C), jnp.float32),
        jax.ShapeDtypeStruct((NC, N, C), jnp.float32),
    ),
    mesh=_mesh,
    compiler_params=pltpu.CompilerParams(use_tc_tiling_on_sc=False, needs_layout_passes=False),
    scratch_types=[
        pltpu.VMEM((CH_BASE + 1, CHUNK), jnp.int32),
        pltpu.VMEM((CH_BASE + 1, CHUNK), jnp.int32),
        pltpu.VMEM((2, CHUNK, C), jnp.float32),
        pltpu.VMEM((2, CHUNK, C), jnp.float32),
        pltpu.VMEM((2, C, CHUNK), jnp.float32),
        pltpu.VMEM((2, CHUNK, C), jnp.float32),
        pltpu.VMEM((ROWS_MAIN, C), jnp.float32),
        pltpu.VMEM_SHARED((N, C), jnp.float32),
        pltpu.SemaphoreType.DMA,
        pltpu.SemaphoreType.DMA,
    ],
)
def _sc_pass1(src_hbm, dst_hbm, el_hbm, er_hbm, eft_hbm,
              ex_hbm, spart_hbm,
              idx_s, idx_d, elb, erb, efb, exb, srow, s_sh, sem0, sem1):
    cid = lax.axis_index("c")
    sid = lax.axis_index("s")
    wid = sid * NC + cid
    start = CH_BASE * wid + jnp.minimum(wid, CH_EXTRA)
    n_w = CH_BASE + jnp.where(wid < CH_EXTRA, 1, 0)

    def zero_body(j, _):
        srow[j] = jnp.zeros((C,), jnp.float32)
        return 0

    lax.fori_loop(0, ROWS_MAIN, zero_body, 0, unroll=8)
    pltpu.sync_copy(srow, s_sh.at[pl.ds(sid * ROWS_MAIN, ROWS_MAIN)])

    @pl.when(sid == 0)
    def _():
        pltpu.sync_copy(srow.at[pl.ds(0, TAIL_ROWS)],
                        s_sh.at[pl.ds(TAIL_BASE, TAIL_ROWS)])

    pltpu.sync_copy(src_hbm.at[pl.ds(start, CH_BASE)], idx_s.at[pl.ds(0, CH_BASE)])
    pltpu.sync_copy(dst_hbm.at[pl.ds(start, CH_BASE)], idx_d.at[pl.ds(0, CH_BASE)])

    @pl.when(wid < CH_EXTRA)
    def _():
        pltpu.sync_copy(src_hbm.at[pl.ds(start + CH_BASE, 1)],
                        idx_s.at[pl.ds(CH_BASE, 1)])
        pltpu.sync_copy(dst_hbm.at[pl.ds(start + CH_BASE, 1)],
                        idx_d.at[pl.ds(CH_BASE, 1)])

    plsc.subcore_barrier()

    rows16 = lax.broadcasted_iota(jnp.int32, (C,), 0)

    def issue(j, slot, sem):
        base = (start + j) * CHUNK
        pltpu.async_copy(el_hbm.at[idx_s.at[j]], elb.at[slot], sem)
        pltpu.async_copy(er_hbm.at[idx_d.at[j]], erb.at[slot], sem)
        pltpu.async_copy(eft_hbm.at[pl.ds(0, C), pl.ds(base, CHUNK)],
                         efb.at[slot], sem)

    def wait_in(slot, sem):
        pltpu.make_async_copy(el_hbm.at[idx_s.at[0]], elb.at[slot], sem).wait()
        pltpu.make_async_copy(er_hbm.at[idx_d.at[0]], erb.at[slot], sem).wait()
        pltpu.make_async_copy(eft_hbm.at[pl.ds(0, C), pl.ds(0, CHUNK)],
                              efb.at[slot], sem).wait()

    def process(j, slot):
        def row_body(r, _):
            v = elb[slot, r] + erb[slot, r]
            v = jnp.where(v > 0, v, SLOPE * v)
            efv = plsc.load_gather(efb.at[slot],
                                   [rows16, jnp.full((C,), r, jnp.int32)])
            exb[slot, r] = jnp.exp(v * efv)
            return 0

        lax.fori_loop(0, CHUNK, row_body, 0, unroll=8)
        pltpu.sync_copy(exb.at[slot], s_sh.at[idx_d.at[j]], add=True)
        pltpu.sync_copy(exb.at[slot], ex_hbm.at[pl.ds((start + j) * CHUNK, CHUNK)])

    issue(0, 0, sem0)

    def pair_body(p, _):
        i0 = 2 * p
        i1 = i0 + 1

        @pl.when(i0 < n_w)
        def _():
            @pl.when(i1 < n_w)
            def _():
                issue(i1, 1, sem1)

            wait_in(0, sem0)
            process(i0, 0)

        @pl.when(i1 < n_w)
        def _():
            @pl.when(i1 + 1 < n_w)
            def _():
                issue(i1 + 1, 0, sem0)

            wait_in(1, sem1)
            process(i1, 1)

        return 0

    lax.fori_loop(0, PAIRS, pair_body, 0)
    plsc.subcore_barrier()

    pltpu.sync_copy(s_sh.at[pl.ds(sid * ROWS_MAIN, ROWS_MAIN)], srow)
    pltpu.sync_copy(srow, spart_hbm.at[cid, pl.ds(sid * ROWS_MAIN, ROWS_MAIN)])

    @pl.when(sid == 0)
    def _():
        pltpu.sync_copy(s_sh.at[pl.ds(TAIL_BASE, TAIL_ROWS)],
                        srow.at[pl.ds(0, TAIL_ROWS)])
        pltpu.sync_copy(srow.at[pl.ds(0, TAIL_ROWS)],
                        spart_hbm.at[cid, pl.ds(TAIL_BASE, TAIL_ROWS)])


@functools.partial(
    pl.kernel,
    out_type=jax.ShapeDtypeStruct((C, E), jnp.float32),
    mesh=_mesh,
    compiler_params=pltpu.CompilerParams(use_tc_tiling_on_sc=False, needs_layout_passes=False),
    scratch_types=[
        pltpu.VMEM((CH_BASE + 1, CHUNK), jnp.int32),
        pltpu.VMEM((2, CHUNK, C), jnp.float32),
        pltpu.VMEM((2, CHUNK, C), jnp.float32),
        pltpu.VMEM((2, CHUNK, C), jnp.float32),
        pltpu.VMEM((2, C, CHUNK), jnp.float32),
        pltpu.SemaphoreType.DMA,
        pltpu.SemaphoreType.DMA,
    ],
)
def _sc_pass2(dst_hbm, ex_hbm, s0_hbm, s1_hbm, out_hbm,
              idx_d, exb, s0b, s1b, outb, sem0, sem1):
    cid = lax.axis_index("c")
    sid = lax.axis_index("s")
    wid = sid * NC + cid
    start = CH_BASE * wid + jnp.minimum(wid, CH_EXTRA)
    n_w = CH_BASE + jnp.where(wid < CH_EXTRA, 1, 0)

    pltpu.sync_copy(dst_hbm.at[pl.ds(start, CH_BASE)], idx_d.at[pl.ds(0, CH_BASE)])

    @pl.when(wid < CH_EXTRA)
    def _():
        pltpu.sync_copy(dst_hbm.at[pl.ds(start + CH_BASE, 1)],
                        idx_d.at[pl.ds(CH_BASE, 1)])

    def issue(j, slot, sem):
        base = (start + j) * CHUNK
        pltpu.async_copy(s0_hbm.at[idx_d.at[j]], s0b.at[slot], sem)
        pltpu.async_copy(s1_hbm.at[idx_d.at[j]], s1b.at[slot], sem)
        pltpu.async_copy(ex_hbm.at[pl.ds(base, CHUNK)], exb.at[slot], sem)

    def wait_in(slot, sem):
        pltpu.make_async_copy(s0_hbm.at[idx_d.at[0]], s0b.at[slot], sem).wait()
        pltpu.make_async_copy(s1_hbm.at[idx_d.at[0]], s1b.at[slot], sem).wait()
        pltpu.make_async_copy(ex_hbm.at[pl.ds(0, CHUNK)], exb.at[slot], sem).wait()

    rows16 = lax.broadcasted_iota(jnp.int32, (C,), 0)

    def process(j, slot):
        def row_body(r, _):
            v = exb[slot, r] / (s0b[slot, r] + s1b[slot, r])
            plsc.store_scatter(outb.at[slot],
                               [rows16, jnp.full((C,), r, jnp.int32)], v)
            return 0

        lax.fori_loop(0, CHUNK, row_body, 0, unroll=8)
        pltpu.sync_copy(outb.at[slot],
                        out_hbm.at[pl.ds(0, C),
                                   pl.ds((start + j) * CHUNK, CHUNK)])

    issue(0, 0, sem0)

    def pair_body(p, _):
        i0 = 2 * p
        i1 = i0 + 1

        @pl.when(i0 < n_w)
        def _():
            @pl.when(i1 < n_w)
            def _():
                issue(i1, 1, sem1)

            wait_in(0, sem0)
            process(i0, 0)

        @pl.when(i1 < n_w)
        def _():
            @pl.when(i1 + 1 < n_w)
            def _():
                issue(i1 + 1, 0, sem0)

            wait_in(1, sem1)
            process(i1, 1)

        return 0

    lax.fori_loop(0, PAIRS, pair_body, 0)


def kernel(node_feat, edge_index, edge_feat, W_fc, W_edge, attn_l, attn_r):
    src2d = edge_index[0].reshape(NCHUNK, CHUNK)
    dst2d = edge_index[1].reshape(NCHUNK, CHUNK)
    W3 = W_fc.reshape(D_IN, C, D_OUT)
    al = attn_l.reshape(C, D_OUT)
    ar = attn_r.reshape(C, D_OUT)
    el, er, eft = _tc_prep(node_feat, W3, al, ar, edge_feat.T, W_edge)
    ex, spart = _sc_pass1(src2d, dst2d, el, er, eft)
    a_t = _sc_pass2(dst2d, ex, spart[0], spart[1])
    return a_t.T.reshape(E, C, 1)
